# Initial kernel scaffold; baseline (speedup 1.0000x reference)
import jax
import jax.numpy as jnp
from jax.experimental import pallas as pl


def kernel(attention_weights, gradient_importance, attention_weight, gradient_weight):
    raise NotImplementedError("write your pallas kernel here")


# speedup vs baseline: 3.7937x; 3.7937x over previous
import functools

import jax
import jax.numpy as jnp
from jax import lax
from jax.experimental import pallas as pl
from jax.experimental.pallas import tpu as pltpu
from jax.experimental.pallas import tpu_sc as plsc

ROWS = 64
COLS = 8192
L = 16
NVEC = COLS // L
UNROLL = 4
K = 4096
NLEV = 5
NBINS = 64

_info = plsc.get_sparse_core_info()
_NC, _NS = _info.num_cores, _info.num_subcores
_NW = _NC * _NS


def _row_kernel(av, gv, comb, hist, w0, w1, lane_iota):
    ones = jnp.ones((L,), jnp.int32)
    lane_base = lane_iota * NBINS
    zero_v = jnp.zeros((L,), jnp.float32)

    def a_body(i, carry):
        ma, mg = carry
        for u in range(UNROLL):
            s = pl.ds((i * UNROLL + u) * L, L)
            ma = jnp.maximum(ma, av[s])
            mg = jnp.maximum(mg, gv[s])
        return ma, mg

    ma, mg = lax.fori_loop(0, NVEC // UNROLL, a_body, (zero_v, zero_v))
    ca = jnp.full((L,), w0, jnp.float32) / (
        jnp.full((L,), jnp.max(ma), jnp.float32) + 1e-8)
    cg = jnp.full((L,), w1, jnp.float32) / (
        jnp.full((L,), jnp.max(mg), jnp.float32) + 1e-8)

    def clr(i, _):
        for u in range(UNROLL):
            hist[pl.ds((i * UNROLL + u) * L, L)] = jnp.zeros((L,), jnp.int32)
        return 0

    lax.fori_loop(0, (NBINS * L) // (L * UNROLL), clr, 0)

    def b_body(i, mc):
        for u in range(UNROLL):
            s = pl.ds((i * UNROLL + u) * L, L)
            c = av[s] * ca + gv[s] * cg
            comb[s] = c
            mc = jnp.maximum(mc, c)
            key = plsc.bitcast(c, jnp.int32)
            d = lax.shift_right_logical(key, 24)
            plsc.addupdate_scatter(hist, [d + lane_base], ones)
        return mc

    mc = lax.fori_loop(0, NVEC // UNROLL, b_body, zero_v)
    maxc = jnp.max(mc)

    def scan_hist(cnt_below):
        acc = jnp.int32(0)
        d_sel = jnp.int32(0)
        exc = jnp.int32(0)
        done = jnp.bool_(False)
        for v in range(NBINS // L):
            tot = hist[pl.ds(v * L, L)]
            for lane in range(1, L):
                tot = tot + hist[pl.ds(lane * NBINS + v * L, L)]
            cum = plsc.cumsum(tot) + acc
            m = cum > K - cnt_below
            f = plsc.all_reduce_ffs(m)
            hit = jnp.logical_and(f < L, jnp.logical_not(done))
            cum_at = jnp.sum(jnp.where(lane_iota == f, cum, 0))
            h_at = jnp.sum(jnp.where(lane_iota == f, tot, 0))
            d_sel = jnp.where(hit, v * L + f, d_sel)
            exc = jnp.where(hit, cum_at - h_at, exc)
            done = jnp.logical_or(done, hit)
            acc = acc + jnp.sum(tot)
        return d_sel, cnt_below + exc

    d0, cnt_below = scan_hist(jnp.int32(0))
    prefix = d0
    for lev in range(1, NLEV):
        sh = 24 - 6 * lev

        def clr2(i, _):
            for u in range(UNROLL):
                hist[pl.ds((i * UNROLL + u) * L, L)] = jnp.zeros((L,), jnp.int32)
            return 0

        lax.fori_loop(0, (NBINS * L) // (L * UNROLL), clr2, 0)

        def c_body(i, _, sh=sh, prefix=prefix):
            for u in range(UNROLL):
                s = pl.ds((i * UNROLL + u) * L, L)
                key = plsc.bitcast(comb[s], jnp.int32)
                match = lax.shift_right_logical(key, sh + 6) == prefix
                d = lax.bitwise_and(lax.shift_right_logical(key, sh),
                                    jnp.int32(NBINS - 1))
                plsc.addupdate_scatter(hist, [d + lane_base], ones, mask=match)
            return 0

        lax.fori_loop(0, NVEC // UNROLL, c_body, 0)
        d, cnt_below = scan_hist(cnt_below)
        prefix = prefix * NBINS + d

    t_vec = plsc.bitcast(jnp.full((L,), prefix, jnp.int32), jnp.float32)
    t = jnp.max(t_vec)
    masked_max = jnp.where(maxc > t, maxc, 0.0)
    s_scale = jnp.ones((L,), jnp.float32) / (
        jnp.full((L,), masked_max, jnp.float32) + 1e-8)

    def d_body(i, _):
        for u in range(UNROLL):
            s = pl.ds((i * UNROLL + u) * L, L)
            c = comb[s]
            comb[s] = jnp.where(c > t_vec, c * s_scale, 0.0)
        return 0

    lax.fori_loop(0, NVEC // UNROLL, d_body, 0)


def _sc_body(a_hbm, g_hbm, p_hbm, out_hbm, av, gv, comb, hist, prm):
    wid = lax.axis_index("s") * _NC + lax.axis_index("c")
    pltpu.sync_copy(p_hbm, prm)
    pv = prm[pl.ds(0, L)]
    w0 = pv[0]
    w1 = pv[1]
    lane_iota = lax.iota(jnp.int32, L)
    for rr in range(ROWS // _NW):
        row = wid * (ROWS // _NW) + rr
        pltpu.sync_copy(a_hbm.at[row], av)
        pltpu.sync_copy(g_hbm.at[row], gv)
        _row_kernel(av, gv, comb, hist, w0, w1, lane_iota)
        pltpu.sync_copy(comb, out_hbm.at[row])


_mesh = plsc.VectorSubcoreMesh(core_axis_name="c", subcore_axis_name="s")

_sc_call = functools.partial(
    pl.kernel,
    mesh=_mesh,
    compiler_params=pltpu.CompilerParams(needs_layout_passes=False),
    out_type=jax.ShapeDtypeStruct((ROWS, COLS), jnp.float32),
    scratch_types=[
        pltpu.VMEM((COLS,), jnp.float32),
        pltpu.VMEM((COLS,), jnp.float32),
        pltpu.VMEM((COLS,), jnp.float32),
        pltpu.VMEM((NBINS * L,), jnp.int32),
        pltpu.VMEM((L,), jnp.float32),
    ],
)(_sc_body)


@jax.jit
def kernel(attention_weights, gradient_importance, attention_weight,
           gradient_weight):
    w = jax.nn.softmax(jnp.stack([attention_weight, gradient_weight]))
    prm = jnp.zeros((L,), jnp.float32).at[0].set(w[0]).at[1].set(w[1])
    return _sc_call(attention_weights, gradient_importance, prm)


# speedup vs baseline: 4.3980x; 1.1593x over previous
import functools

import jax
import jax.numpy as jnp
from jax import lax
from jax.experimental import pallas as pl
from jax.experimental.pallas import tpu as pltpu
from jax.experimental.pallas import tpu_sc as plsc

ROWS = 64
COLS = 8192
L = 16
NVEC = COLS // L
UNROLL = 4
K = 4096
NLEV = 5
NBINS = 64

_info = plsc.get_sparse_core_info()
_NC, _NS = _info.num_cores, _info.num_subcores
_NW = _NC * _NS


def _row_kernel(av, gv, comb, hist, keys, w0, w1, lane_iota):
    ones = jnp.ones((L,), jnp.int32)
    lane_base = lane_iota * NBINS
    zero_v = jnp.zeros((L,), jnp.float32)

    def a_body(i, carry):
        ma, mg = carry
        for u in range(UNROLL):
            s = pl.ds((i * UNROLL + u) * L, L)
            ma = jnp.maximum(ma, av[s])
            mg = jnp.maximum(mg, gv[s])
        return ma, mg

    ma, mg = lax.fori_loop(0, NVEC // UNROLL, a_body, (zero_v, zero_v))
    ca = jnp.full((L,), w0, jnp.float32) / (
        jnp.full((L,), jnp.max(ma), jnp.float32) + 1e-8)
    cg = jnp.full((L,), w1, jnp.float32) / (
        jnp.full((L,), jnp.max(mg), jnp.float32) + 1e-8)

    def clr(i, _):
        for u in range(UNROLL):
            hist[pl.ds((i * UNROLL + u) * L, L)] = jnp.zeros((L,), jnp.int32)
        return 0

    lax.fori_loop(0, (NBINS * L) // (L * UNROLL), clr, 0)

    def b_body(i, mc):
        for u in range(UNROLL):
            s = pl.ds((i * UNROLL + u) * L, L)
            c = av[s] * ca + gv[s] * cg
            comb[s] = c
            mc = jnp.maximum(mc, c)
            key = plsc.bitcast(c, jnp.int32)
            d = lax.shift_right_logical(key, 24)
            plsc.addupdate_scatter(hist, [d + lane_base], ones)
        return mc

    mc = lax.fori_loop(0, NVEC // UNROLL, b_body, zero_v)
    maxc = jnp.max(mc)

    def scan_hist(cnt_below):
        acc = jnp.int32(0)
        d_sel = jnp.int32(0)
        exc = jnp.int32(0)
        done = jnp.bool_(False)
        for v in range(NBINS // L):
            tot = hist[pl.ds(v * L, L)]
            for lane in range(1, L):
                tot = tot + hist[pl.ds(lane * NBINS + v * L, L)]
            cum = plsc.cumsum(tot) + acc
            m = cum > K - cnt_below
            f = plsc.all_reduce_ffs(m)
            hit = jnp.logical_and(f < L, jnp.logical_not(done))
            cum_at = jnp.sum(jnp.where(lane_iota == f, cum, 0))
            h_at = jnp.sum(jnp.where(lane_iota == f, tot, 0))
            d_sel = jnp.where(hit, v * L + f, d_sel)
            exc = jnp.where(hit, cum_at - h_at, exc)
            done = jnp.logical_or(done, hit)
            acc = acc + jnp.sum(tot)
        return d_sel, cnt_below + exc

    d0, cnt_below = scan_hist(jnp.int32(0))
    prefix = d0

    def clr2(i, _):
        for u in range(UNROLL):
            hist[pl.ds((i * UNROLL + u) * L, L)] = jnp.zeros((L,), jnp.int32)
        return 0

    lax.fori_loop(0, (NBINS * L) // (L * UNROLL), clr2, 0)

    def c1_body(i, off):
        for u in range(UNROLL):
            s = pl.ds((i * UNROLL + u) * L, L)
            key = plsc.bitcast(comb[s], jnp.int32)
            match = lax.shift_right_logical(key, 24) == prefix
            d = lax.bitwise_and(lax.shift_right_logical(key, 18),
                                jnp.int32(NBINS - 1))
            plsc.addupdate_scatter(hist, [d + lane_base], ones, mask=match)
            plsc.store_compressed(keys.at[pl.ds(off, L)], key, mask=match)
            off = off + plsc.all_reduce_population_count(match)[0]
        return off

    m = lax.fori_loop(0, NVEC // UNROLL, c1_body, jnp.int32(0))
    keys[pl.ds(m, L)] = jnp.full((L,), jnp.int32(0x7FFFFFFF))
    nv2 = lax.shift_right_logical(m + (L - 1), 4)
    d, cnt_below = scan_hist(cnt_below)
    prefix = prefix * NBINS + d

    for lev in range(2, NLEV):
        sh = 24 - 6 * lev

        def clr3(i, _):
            for u in range(UNROLL):
                hist[pl.ds((i * UNROLL + u) * L, L)] = jnp.zeros((L,), jnp.int32)
            return 0

        lax.fori_loop(0, (NBINS * L) // (L * UNROLL), clr3, 0)

        def c_body(i, _, sh=sh, prefix=prefix):
            key = keys[pl.ds(i * L, L)]
            match = lax.shift_right_logical(key, sh + 6) == prefix
            d = lax.bitwise_and(lax.shift_right_logical(key, sh),
                                jnp.int32(NBINS - 1))
            plsc.addupdate_scatter(hist, [d + lane_base], ones, mask=match)
            return 0

        lax.fori_loop(0, nv2, c_body, 0)
        d, cnt_below = scan_hist(cnt_below)
        prefix = prefix * NBINS + d

    t_vec = plsc.bitcast(jnp.full((L,), prefix, jnp.int32), jnp.float32)
    t = jnp.max(t_vec)
    masked_max = jnp.where(maxc > t, maxc, 0.0)
    s_scale = jnp.ones((L,), jnp.float32) / (
        jnp.full((L,), masked_max, jnp.float32) + 1e-8)

    def d_body(i, _):
        for u in range(UNROLL):
            s = pl.ds((i * UNROLL + u) * L, L)
            c = comb[s]
            comb[s] = jnp.where(c > t_vec, c * s_scale, 0.0)
        return 0

    lax.fori_loop(0, NVEC // UNROLL, d_body, 0)


def _sc_body(a_hbm, g_hbm, p_hbm, out_hbm, av, gv, comb, hist, keys, prm):
    wid = lax.axis_index("s") * _NC + lax.axis_index("c")
    pltpu.sync_copy(p_hbm, prm)
    pv = prm[pl.ds(0, L)]
    w0 = pv[0]
    w1 = pv[1]
    lane_iota = lax.iota(jnp.int32, L)
    for rr in range(ROWS // _NW):
        row = wid * (ROWS // _NW) + rr
        pltpu.sync_copy(a_hbm.at[row], av)
        pltpu.sync_copy(g_hbm.at[row], gv)
        _row_kernel(av, gv, comb, hist, keys, w0, w1, lane_iota)
        pltpu.sync_copy(comb, out_hbm.at[row])


_mesh = plsc.VectorSubcoreMesh(core_axis_name="c", subcore_axis_name="s")

_sc_call = functools.partial(
    pl.kernel,
    mesh=_mesh,
    compiler_params=pltpu.CompilerParams(needs_layout_passes=False),
    out_type=jax.ShapeDtypeStruct((ROWS, COLS), jnp.float32),
    scratch_types=[
        pltpu.VMEM((COLS,), jnp.float32),
        pltpu.VMEM((COLS,), jnp.float32),
        pltpu.VMEM((COLS,), jnp.float32),
        pltpu.VMEM((NBINS * L,), jnp.int32),
        pltpu.VMEM((COLS + L,), jnp.int32),
        pltpu.VMEM((L,), jnp.float32),
    ],
)(_sc_body)


@jax.jit
def kernel(attention_weights, gradient_importance, attention_weight,
           gradient_weight):
    w = jax.nn.softmax(jnp.stack([attention_weight, gradient_weight]))
    prm = jnp.zeros((L,), jnp.float32).at[0].set(w[0]).at[1].set(w[1])
    return _sc_call(attention_weights, gradient_importance, prm)
